# baseline (speedup 1.0000x reference)
import jax
import jax.numpy as jnp
from jax.experimental import pallas as pl

_E = 64
_K = 8


def _router_kernel(x_ref, w_ref, scores_ref, ew_ref, ei_ref):
    x = x_ref[...]
    w = w_ref[...]
    logits = jax.lax.dot_general(
        x, w, (((1,), (1,)), ((), ())), preferred_element_type=jnp.float32
    )
    m = jnp.max(logits, axis=-1, keepdims=True)
    e = jnp.exp(logits - m)
    s = e / jnp.sum(e, axis=-1, keepdims=True)
    scores_ref[...] = s

    iota = jax.lax.broadcasted_iota(jnp.int32, s.shape, 1)
    val = s
    ew_cols = []
    ei_cols = []
    for _ in range(_K):
        mx = jnp.max(val, axis=-1, keepdims=True)
        idx = jnp.min(jnp.where(val == mx, iota, _E), axis=-1, keepdims=True)
        ew_cols.append(mx)
        ei_cols.append(idx)
        val = jnp.where(iota == idx, -1.0, val)
    ew_ref[...] = jnp.concatenate(ew_cols, axis=1)
    ei_ref[...] = jnp.concatenate(ei_cols, axis=1)


def kernel(x, W):
    sl, bs, hs = x.shape
    m = sl * bs
    x2 = x.reshape(m, hs)
    tm = 256
    scores, ew, ei = pl.pallas_call(
        _router_kernel,
        grid=(m // tm,),
        in_specs=[
            pl.BlockSpec((tm, hs), lambda i: (i, 0)),
            pl.BlockSpec((_E, hs), lambda i: (0, 0)),
        ],
        out_specs=[
            pl.BlockSpec((tm, _E), lambda i: (i, 0)),
            pl.BlockSpec((tm, _K), lambda i: (i, 0)),
            pl.BlockSpec((tm, _K), lambda i: (i, 0)),
        ],
        out_shape=[
            jax.ShapeDtypeStruct((m, _E), jnp.float32),
            jax.ShapeDtypeStruct((m, _K), jnp.float32),
            jax.ShapeDtypeStruct((m, _K), jnp.int32),
        ],
    )(x2, W)
    return scores, ew, ei, jnp.float32(0.0)


# speedup vs baseline: 1.2309x; 1.2309x over previous
import jax
import jax.numpy as jnp
from jax.experimental import pallas as pl

_E = 64
_K = 8


def _router_kernel(x_ref, w_ref, scores_ref, ew_ref, ei_ref):
    x = x_ref[...]
    w = w_ref[...]
    lt = jax.lax.dot_general(
        w, x, (((1,), (1,)), ((), ())), preferred_element_type=jnp.float32
    )
    m = jnp.max(lt, axis=0, keepdims=True)
    e = jnp.exp(lt - m)
    s = e / jnp.sum(e, axis=0, keepdims=True)
    scores_ref[...] = s

    iota = jax.lax.broadcasted_iota(jnp.int32, s.shape, 0)
    val = s
    ew_rows = []
    ei_rows = []
    for _ in range(_K):
        mx = jnp.max(val, axis=0, keepdims=True)
        idx = jnp.min(jnp.where(val == mx, iota, _E), axis=0, keepdims=True)
        ew_rows.append(mx)
        ei_rows.append(idx)
        val = jnp.where(iota == idx, -1.0, val)
    ew_ref[...] = jnp.concatenate(ew_rows, axis=0)
    ei_ref[...] = jnp.concatenate(ei_rows, axis=0)


def kernel(x, W):
    sl, bs, hs = x.shape
    m = sl * bs
    x2 = x.reshape(m, hs)
    tm = 256
    scores_t, ew_t, ei_t = pl.pallas_call(
        _router_kernel,
        grid=(m // tm,),
        in_specs=[
            pl.BlockSpec((tm, hs), lambda i: (i, 0)),
            pl.BlockSpec((_E, hs), lambda i: (0, 0)),
        ],
        out_specs=[
            pl.BlockSpec((_E, tm), lambda i: (0, i)),
            pl.BlockSpec((_K, tm), lambda i: (0, i)),
            pl.BlockSpec((_K, tm), lambda i: (0, i)),
        ],
        out_shape=[
            jax.ShapeDtypeStruct((_E, m), jnp.float32),
            jax.ShapeDtypeStruct((_K, m), jnp.float32),
            jax.ShapeDtypeStruct((_K, m), jnp.int32),
        ],
    )(x2, W)
    return scores_t.T, ew_t.T, ei_t.T, jnp.float32(0.0)
